# Initial kernel scaffold; baseline (speedup 1.0000x reference)
import jax
import jax.numpy as jnp
from jax.experimental import pallas as pl


def kernel(X, adj_mat, W0, b0, W1, b1, W2, b2, ln1_g, ln1_b, ln2_g, ln2_b):
    raise NotImplementedError("write your pallas kernel here")


# speedup vs baseline: 49.2945x; 49.2945x over previous
import functools

import jax
import jax.numpy as jnp
from jax.experimental import pallas as pl
from jax.experimental.pallas import tpu as pltpu


def _xw_kernel(x_ref, w_ref, o_ref):
    o_ref[...] = jax.lax.dot(
        x_ref[...], w_ref[...],
        precision=jax.lax.Precision.HIGHEST,
        preferred_element_type=jnp.float32,
    )


def _mp_kernel(use_ln, num_k, adj_ref, xw_ref, res_ref, b_ref, g_ref, bb_ref,
               o_ref, acc_ref):
    k = pl.program_id(1)

    @pl.when(k == 0)
    def _():
        acc_ref[...] = jnp.zeros_like(acc_ref)

    a = adj_ref[...].astype(jnp.bfloat16)
    xw = xw_ref[...].astype(jnp.bfloat16)
    acc_ref[...] += jax.lax.dot_general(
        a, xw, (((0,), (0,)), ((), ())),
        preferred_element_type=jnp.float32,
    )

    @pl.when(k == num_k - 1)
    def _():
        t = acc_ref[...] + b_ref[...]
        if use_ln:
            t = t + res_ref[...]
            mu = jnp.mean(t, axis=-1, keepdims=True)
            var = jnp.mean((t - mu) ** 2, axis=-1, keepdims=True)
            t = (t - mu) * jax.lax.rsqrt(var + 1e-5) * g_ref[...] + bb_ref[...]
        o_ref[...] = jnp.maximum(t, 0.0)


def _layer(adj, x, W, b, res, g, bb, use_ln):
    n, d = x.shape
    h = W.shape[1]
    rb = 2000 if n % 2000 == 0 else n
    xw = pl.pallas_call(
        _xw_kernel,
        grid=(n // rb,),
        in_specs=[
            pl.BlockSpec((rb, d), lambda i: (i, 0)),
            pl.BlockSpec((d, h), lambda i: (0, 0)),
        ],
        out_specs=pl.BlockSpec((rb, h), lambda i: (i, 0)),
        out_shape=jax.ShapeDtypeStruct((n, h), jnp.float32),
    )(x, W)

    bd = 1024
    ks = 2000 if n % 2000 == 0 else n
    num_j = pl.cdiv(n, bd)
    num_k = n // ks
    out = pl.pallas_call(
        functools.partial(_mp_kernel, use_ln, num_k),
        grid=(num_j, num_k),
        in_specs=[
            pl.BlockSpec((ks, bd), lambda j, k: (k, j)),
            pl.BlockSpec((ks, h), lambda j, k: (k, 0)),
            pl.BlockSpec((bd, h), lambda j, k: (j, 0)),
            pl.BlockSpec((1, h), lambda j, k: (0, 0)),
            pl.BlockSpec((1, h), lambda j, k: (0, 0)),
            pl.BlockSpec((1, h), lambda j, k: (0, 0)),
        ],
        out_specs=pl.BlockSpec((bd, h), lambda j, k: (j, 0)),
        out_shape=jax.ShapeDtypeStruct((n, h), jnp.float32),
        scratch_shapes=[pltpu.VMEM((bd, h), jnp.float32)],
        compiler_params=pltpu.CompilerParams(
            dimension_semantics=("parallel", "arbitrary")),
    )(adj, xw, res, b.reshape(1, h), g.reshape(1, h), bb.reshape(1, h))
    return out


def kernel(X, adj_mat, W0, b0, W1, b1, W2, b2, ln1_g, ln1_b, ln2_g, ln2_b):
    h1 = _layer(adj_mat, X, W0, b0, X, ln1_g, ln1_b, use_ln=False)
    h2 = _layer(adj_mat, h1, W1, b1, h1, ln1_g, ln1_b, use_ln=True)
    h3 = _layer(adj_mat, h2, W2, b2, h2, ln2_g, ln2_b, use_ln=True)
    return h3


# speedup vs baseline: 50.0084x; 1.0145x over previous
import functools

import jax
import jax.numpy as jnp
from jax.experimental import pallas as pl
from jax.experimental.pallas import tpu as pltpu


def _xw_kernel(x_ref, w_ref, o_ref):
    o_ref[...] = jax.lax.dot(
        x_ref[...], w_ref[...],
        precision=jax.lax.Precision.HIGHEST,
        preferred_element_type=jnp.float32,
    )


def _mp_kernel(use_ln, emit_bf16, num_k, adj_ref, xw_ref, res_ref, b_ref,
               g_ref, bb_ref, *rest):
    if emit_bf16:
        o_ref, abf_ref, acc_ref = rest
    else:
        o_ref, acc_ref = rest
    k = pl.program_id(1)

    @pl.when(k == 0)
    def _():
        acc_ref[...] = jnp.zeros_like(acc_ref)

    a = adj_ref[...].astype(jnp.bfloat16)
    if emit_bf16:
        abf_ref[...] = a
    xw = xw_ref[...].astype(jnp.bfloat16)
    acc_ref[...] += jax.lax.dot_general(
        a, xw, (((0,), (0,)), ((), ())),
        preferred_element_type=jnp.float32,
    )

    @pl.when(k == num_k - 1)
    def _():
        t = acc_ref[...] + b_ref[...]
        if use_ln:
            t = t + res_ref[...]
            mu = jnp.mean(t, axis=-1, keepdims=True)
            var = jnp.mean((t - mu) ** 2, axis=-1, keepdims=True)
            t = (t - mu) * jax.lax.rsqrt(var + 1e-5) * g_ref[...] + bb_ref[...]
        o_ref[...] = jnp.maximum(t, 0.0)


def _layer(adj, x, W, b, res, g, bb, use_ln, emit_bf16=False):
    n, d = x.shape
    h = W.shape[1]
    rb = 2000 if n % 2000 == 0 else n
    xw = pl.pallas_call(
        _xw_kernel,
        grid=(n // rb,),
        in_specs=[
            pl.BlockSpec((rb, d), lambda i: (i, 0)),
            pl.BlockSpec((d, h), lambda i: (0, 0)),
        ],
        out_specs=pl.BlockSpec((rb, h), lambda i: (i, 0)),
        out_shape=jax.ShapeDtypeStruct((n, h), jnp.float32),
    )(x, W)

    bd = 1024
    ks = 2000 if n % 2000 == 0 else n
    num_j = pl.cdiv(n, bd)
    num_k = n // ks
    out_shapes = [jax.ShapeDtypeStruct((n, h), jnp.float32)]
    out_specs = [pl.BlockSpec((bd, h), lambda j, k: (j, 0))]
    if emit_bf16:
        out_shapes.append(jax.ShapeDtypeStruct(adj.shape, jnp.bfloat16))
        out_specs.append(pl.BlockSpec((ks, bd), lambda j, k: (k, j)))
    out = pl.pallas_call(
        functools.partial(_mp_kernel, use_ln, emit_bf16, num_k),
        grid=(num_j, num_k),
        in_specs=[
            pl.BlockSpec((ks, bd), lambda j, k: (k, j)),
            pl.BlockSpec((ks, h), lambda j, k: (k, 0)),
            pl.BlockSpec((bd, h), lambda j, k: (j, 0)),
            pl.BlockSpec((1, h), lambda j, k: (0, 0)),
            pl.BlockSpec((1, h), lambda j, k: (0, 0)),
            pl.BlockSpec((1, h), lambda j, k: (0, 0)),
        ],
        out_specs=out_specs,
        out_shape=out_shapes,
        scratch_shapes=[pltpu.VMEM((bd, h), jnp.float32)],
        compiler_params=pltpu.CompilerParams(
            dimension_semantics=("parallel", "arbitrary")),
    )(adj, xw, res, b.reshape(1, h), g.reshape(1, h), bb.reshape(1, h))
    return out


def kernel(X, adj_mat, W0, b0, W1, b1, W2, b2, ln1_g, ln1_b, ln2_g, ln2_b):
    h1, adj_bf = _layer(adj_mat, X, W0, b0, X, ln1_g, ln1_b,
                        use_ln=False, emit_bf16=True)
    h2, = _layer(adj_bf, h1, W1, b1, h1, ln1_g, ln1_b, use_ln=True)
    h3, = _layer(adj_bf, h2, W2, b2, h2, ln2_g, ln2_b, use_ln=True)
    return h3


# speedup vs baseline: 53.5610x; 1.0710x over previous
import functools

import jax
import jax.numpy as jnp
from jax.experimental import pallas as pl
from jax.experimental.pallas import tpu as pltpu


def _xwt_kernel(wt_ref, xt_ref, o_ref):
    o_ref[...] = jax.lax.dot(
        wt_ref[...].astype(jnp.bfloat16), xt_ref[...].astype(jnp.bfloat16),
        preferred_element_type=jnp.float32,
    ).astype(jnp.bfloat16)


def _mpt_kernel(use_ln, emit_bf16, fuse_next, num_k, ks, n,
                adj_ref, xwt_ref, rest_ref, b_ref, g_ref, bb_ref, wnt_ref,
                *outs):
    refs = list(outs)
    ht_ref = refs.pop(0)
    abf_ref = refs.pop(0) if emit_bf16 else None
    xwnt_ref = refs.pop(0) if fuse_next else None
    acc_ref = refs.pop(0)
    j = pl.program_id(0)
    k = pl.program_id(1)

    @pl.when(k == 0)
    def _():
        acc_ref[...] = jnp.zeros_like(acc_ref)

    a = adj_ref[...].astype(jnp.bfloat16)
    if emit_bf16:
        rows = k * ks + jax.lax.broadcasted_iota(jnp.int32, (ks, 1), 0)
        a = jnp.where(rows < n, a, jnp.bfloat16(0))
        abf_ref[...] = a
    xwt = xwt_ref[:, pl.ds(k * ks, ks)]
    acc_ref[...] += jax.lax.dot(
        xwt, a, preferred_element_type=jnp.float32)

    @pl.when(k == num_k - 1)
    def _():
        t = acc_ref[...] + b_ref[...]
        if use_ln:
            t = t + rest_ref[...]
            mu = jnp.mean(t, axis=0, keepdims=True)
            var = jnp.mean((t - mu) ** 2, axis=0, keepdims=True)
            t = (t - mu) * jax.lax.rsqrt(var + 1e-5) * g_ref[...] + bb_ref[...]
        ht = jnp.maximum(t, 0.0)
        ht_ref[...] = ht
        if fuse_next:
            bd = ht_ref.shape[1]
            xwnt = jax.lax.dot(
                wnt_ref[...].astype(jnp.bfloat16), ht.astype(jnp.bfloat16),
                preferred_element_type=jnp.float32,
            ).astype(jnp.bfloat16)
            col = j * bd + jax.lax.broadcasted_iota(jnp.int32, (1, bd), 1)
            xwnt_ref[...] = jnp.where(col < n, xwnt, jnp.bfloat16(0))


def _layer(adj, xwt, rest, b, g, bb, wnt, use_ln, emit_bf16, fuse_next,
           bd=1024, ks=2048):
    n = adj.shape[1]
    h, np_ = xwt.shape
    num_j = pl.cdiv(n, bd)
    num_k = np_ // ks
    out_shapes = [jax.ShapeDtypeStruct((h, n), jnp.float32)]
    out_specs = [pl.BlockSpec((h, bd), lambda j, k: (0, j))]
    if emit_bf16:
        out_shapes.append(jax.ShapeDtypeStruct((np_, n), jnp.bfloat16))
        out_specs.append(pl.BlockSpec((ks, bd), lambda j, k: (k, j)))
    if fuse_next:
        out_shapes.append(jax.ShapeDtypeStruct((h, np_), jnp.bfloat16))
        out_specs.append(pl.BlockSpec((h, bd), lambda j, k: (0, j)))
    outs = pl.pallas_call(
        functools.partial(
            _mpt_kernel, use_ln, emit_bf16, fuse_next, num_k, ks, n),
        grid=(num_j, num_k),
        in_specs=[
            pl.BlockSpec((ks, bd), lambda j, k: (k, j)),
            pl.BlockSpec((h, np_), lambda j, k: (0, 0)),
            pl.BlockSpec((h, bd), lambda j, k: (0, j)),
            pl.BlockSpec((h, 1), lambda j, k: (0, 0)),
            pl.BlockSpec((h, 1), lambda j, k: (0, 0)),
            pl.BlockSpec((h, 1), lambda j, k: (0, 0)),
            pl.BlockSpec((h, h), lambda j, k: (0, 0)),
        ],
        out_specs=out_specs,
        out_shape=out_shapes,
        scratch_shapes=[pltpu.VMEM((h, bd), jnp.float32)],
        compiler_params=pltpu.CompilerParams(
            dimension_semantics=("parallel", "arbitrary")),
    )(adj, xwt, rest, b.reshape(h, 1), g.reshape(h, 1), bb.reshape(h, 1), wnt)
    return outs


def kernel(X, adj_mat, W0, b0, W1, b1, W2, b2, ln1_g, ln1_b, ln2_g, ln2_b):
    n, d = X.shape
    h = W0.shape[1]
    ks = 2048
    np_ = ((n + ks - 1) // ks) * ks
    xt = jnp.pad(X.T, ((0, 0), (0, np_ - n)))
    xw1t = pl.pallas_call(
        _xwt_kernel,
        grid=(1,),
        in_specs=[
            pl.BlockSpec((h, d), lambda i: (0, 0)),
            pl.BlockSpec((d, np_), lambda i: (0, 0)),
        ],
        out_specs=pl.BlockSpec((h, np_), lambda i: (0, 0)),
        out_shape=jax.ShapeDtypeStruct((h, np_), jnp.bfloat16),
    )(W0.T, xt)

    h1t, adj_bf, xw2t = _layer(adj_mat, xw1t, xt, b0, ln1_g, ln1_b, W1.T,
                               use_ln=False, emit_bf16=True, fuse_next=True)
    h2t, xw3t = _layer(adj_bf, xw2t, h1t, b1, ln1_g, ln1_b, W2.T,
                       use_ln=True, emit_bf16=False, fuse_next=True)
    h3t, = _layer(adj_bf, xw3t, h2t, b2, ln2_g, ln2_b, W2.T,
                  use_ln=True, emit_bf16=False, fuse_next=False)
    return h3t.T


# speedup vs baseline: 59.3338x; 1.1078x over previous
import functools

import jax
import jax.numpy as jnp
from jax.experimental import pallas as pl
from jax.experimental.pallas import tpu as pltpu


def _mpt_kernel(use_ln, fuse_next, layer1, num_k, ks, n, *refs):
    refs = list(refs)
    adj_ref = refs.pop(0)
    if layer1:
        x_ref = refs.pop(0)
        w0_ref = refs.pop(0)
    else:
        xwt_ref = refs.pop(0)
    rest_ref = refs.pop(0) if use_ln else None
    b_ref = refs.pop(0)
    if use_ln:
        g_ref = refs.pop(0)
        bb_ref = refs.pop(0)
    wnt_ref = refs.pop(0) if fuse_next else None
    ht_ref = refs.pop(0)
    abf_ref = refs.pop(0) if layer1 else None
    xwnt_ref = refs.pop(0) if fuse_next else None
    acc_ref = refs.pop(0)
    xwt_s = refs.pop(0) if layer1 else None

    j = pl.program_id(0)
    k = pl.program_id(1)
    np_ = num_k * ks

    if layer1:
        @pl.when((j == 0) & (k == 0))
        def _():
            v = jax.lax.dot_general(
                w0_ref[...], x_ref[...], (((0,), (1,)), ((), ())),
                preferred_element_type=jnp.float32,
            )
            col = jax.lax.broadcasted_iota(jnp.int32, (1, np_), 1)
            xwt_s[...] = jnp.where(col < n, v.astype(jnp.bfloat16),
                                   jnp.bfloat16(0))
        xwt_ref = xwt_s

    @pl.when(k == 0)
    def _():
        acc_ref[...] = jnp.zeros_like(acc_ref)

    a = adj_ref[...].astype(jnp.bfloat16)
    if layer1:
        rows = k * ks + jax.lax.broadcasted_iota(jnp.int32, (ks, 1), 0)
        a = jnp.where(rows < n, a, jnp.bfloat16(0))
        abf_ref[...] = a
    xwt = xwt_ref[:, pl.ds(k * ks, ks)]
    acc_ref[...] += jax.lax.dot(
        xwt, a, preferred_element_type=jnp.float32)

    @pl.when(k == num_k - 1)
    def _():
        t = acc_ref[...] + b_ref[...]
        if use_ln:
            t = t + rest_ref[...]
            mu = jnp.mean(t, axis=0, keepdims=True)
            var = jnp.mean((t - mu) ** 2, axis=0, keepdims=True)
            t = (t - mu) * jax.lax.rsqrt(var + 1e-5) * g_ref[...] + bb_ref[...]
        ht = jnp.maximum(t, 0.0)
        ht_ref[...] = ht
        if fuse_next:
            bd = ht_ref.shape[1]
            xwnt = jax.lax.dot(
                wnt_ref[...].astype(jnp.bfloat16), ht.astype(jnp.bfloat16),
                preferred_element_type=jnp.float32,
            ).astype(jnp.bfloat16)
            col = j * bd + jax.lax.broadcasted_iota(jnp.int32, (1, bd), 1)
            xwnt_ref[...] = jnp.where(col < n, xwnt, jnp.bfloat16(0))


def _layer(adj, x_or_xwt, rest, b, g, bb, wnt, w0, use_ln, fuse_next, layer1,
           bd, ks=2048):
    n = adj.shape[1]
    h = b.shape[0]
    np_ = ((n + ks - 1) // ks) * ks
    num_j = pl.cdiv(n, bd)
    num_k = np_ // ks

    in_arrays = [adj]
    in_specs = [pl.BlockSpec((ks, bd), lambda j, k: (k, j))]
    if layer1:
        d = x_or_xwt.shape[1]
        in_arrays += [x_or_xwt, w0]
        in_specs += [pl.BlockSpec((np_, d), lambda j, k: (0, 0)),
                     pl.BlockSpec((d, h), lambda j, k: (0, 0))]
    else:
        in_arrays.append(x_or_xwt)
        in_specs.append(pl.BlockSpec((h, np_), lambda j, k: (0, 0)))
    if use_ln:
        in_arrays.append(rest)
        in_specs.append(pl.BlockSpec((h, bd), lambda j, k: (0, j)))
    in_arrays.append(b.reshape(h, 1))
    in_specs.append(pl.BlockSpec((h, 1), lambda j, k: (0, 0)))
    if use_ln:
        in_arrays += [g.reshape(h, 1), bb.reshape(h, 1)]
        in_specs += [pl.BlockSpec((h, 1), lambda j, k: (0, 0)),
                     pl.BlockSpec((h, 1), lambda j, k: (0, 0))]
    if fuse_next:
        in_arrays.append(wnt)
        in_specs.append(pl.BlockSpec((h, h), lambda j, k: (0, 0)))

    out_shapes = [jax.ShapeDtypeStruct((h, n), jnp.float32)]
    out_specs = [pl.BlockSpec((h, bd), lambda j, k: (0, j))]
    if layer1:
        out_shapes.append(jax.ShapeDtypeStruct((np_, n), jnp.bfloat16))
        out_specs.append(pl.BlockSpec((ks, bd), lambda j, k: (k, j)))
    if fuse_next:
        out_shapes.append(jax.ShapeDtypeStruct((h, np_), jnp.bfloat16))
        out_specs.append(pl.BlockSpec((h, bd), lambda j, k: (0, j)))

    scratch = [pltpu.VMEM((h, bd), jnp.float32)]
    if layer1:
        scratch.append(pltpu.VMEM((h, np_), jnp.bfloat16))

    return pl.pallas_call(
        functools.partial(
            _mpt_kernel, use_ln, fuse_next, layer1, num_k, ks, n),
        grid=(num_j, num_k),
        in_specs=in_specs,
        out_specs=out_specs,
        out_shape=out_shapes,
        scratch_shapes=scratch,
        compiler_params=pltpu.CompilerParams(
            dimension_semantics=("parallel", "arbitrary")),
    )(*in_arrays)


def kernel(X, adj_mat, W0, b0, W1, b1, W2, b2, ln1_g, ln1_b, ln2_g, ln2_b):
    h1t, adj_bf, xw2t = _layer(
        adj_mat, X, None, b0, None, None, W1.T, W0,
        use_ln=False, fuse_next=True, layer1=True, bd=1024)
    h2t, xw3t = _layer(
        adj_bf, xw2t, h1t, b1, ln1_g, ln1_b, W2.T, None,
        use_ln=True, fuse_next=True, layer1=False, bd=2048)
    h3t, = _layer(
        adj_bf, xw3t, h2t, b2, ln2_g, ln2_b, None, None,
        use_ln=True, fuse_next=False, layer1=False, bd=2048)
    return h3t.T


# speedup vs baseline: 62.0632x; 1.0460x over previous
import functools

import jax
import jax.numpy as jnp
from jax.experimental import pallas as pl
from jax.experimental.pallas import tpu as pltpu


def _layer1_kernel(num_k, ks, n, adj_ref, x_ref, w0_ref, b_ref, wnt_ref,
                   ht_ref, abf_ref, xwnt_ref, acc_ref, xwt_s):
    j = pl.program_id(0)
    k = pl.program_id(1)
    np_ = num_k * ks

    @pl.when((j == 0) & (k == 0))
    def _():
        v = jax.lax.dot_general(
            w0_ref[...], x_ref[...], (((0,), (1,)), ((), ())),
            preferred_element_type=jnp.float32,
        )
        col = jax.lax.broadcasted_iota(jnp.int32, (1, np_), 1)
        xwt_s[...] = jnp.where(col < n, v.astype(jnp.bfloat16),
                               jnp.bfloat16(0))

    @pl.when(k == 0)
    def _():
        acc_ref[...] = jnp.zeros_like(acc_ref)

    a = adj_ref[...].astype(jnp.bfloat16)
    rows = k * ks + jax.lax.broadcasted_iota(jnp.int32, (ks, 1), 0)
    a = jnp.where(rows < n, a, jnp.bfloat16(0))
    abf_ref[...] = a
    acc_ref[...] += jax.lax.dot(
        xwt_s[:, pl.ds(k * ks, ks)], a, preferred_element_type=jnp.float32)

    @pl.when(k == num_k - 1)
    def _():
        ht = jnp.maximum(acc_ref[...] + b_ref[...], 0.0)
        ht_ref[...] = ht
        bd = ht_ref.shape[1]
        xwnt = jax.lax.dot(
            wnt_ref[...].astype(jnp.bfloat16), ht.astype(jnp.bfloat16),
            preferred_element_type=jnp.float32,
        ).astype(jnp.bfloat16)
        col = j * bd + jax.lax.broadcasted_iota(jnp.int32, (1, bd), 1)
        xwnt_ref[...] = jnp.where(col < n, xwnt, jnp.bfloat16(0))


def _ln(t, g, bb):
    mu = jnp.mean(t, axis=0, keepdims=True)
    var = jnp.mean((t - mu) ** 2, axis=0, keepdims=True)
    return (t - mu) * jax.lax.rsqrt(var + 1e-5) * g + bb


def _layer23_kernel(num_k, ks, n, abf_ref, xwt_ref, rest_ref,
                    b1_ref, g1_ref, bb1_ref, b2_ref, g2_ref, bb2_ref, wnt_ref,
                    out_ref, acc_ref, xwt3_s, h2t_s):
    l = pl.program_id(0)
    j = pl.program_id(1)
    k = pl.program_id(2)
    bd = acc_ref.shape[1]

    @pl.when(k == 0)
    def _():
        acc_ref[...] = jnp.zeros_like(acc_ref)

    a = abf_ref[...]

    @pl.when(l == 0)
    def _():
        acc_ref[...] += jax.lax.dot(
            xwt_ref[:, pl.ds(k * ks, ks)], a,
            preferred_element_type=jnp.float32)

    @pl.when(l == 1)
    def _():
        acc_ref[...] += jax.lax.dot(
            xwt3_s[:, pl.ds(k * ks, ks)], a,
            preferred_element_type=jnp.float32)

    @pl.when(k == num_k - 1)
    def _():
        @pl.when(l == 0)
        def _():
            t = acc_ref[...] + b1_ref[...] + rest_ref[...]
            h2 = jnp.maximum(_ln(t, g1_ref[...], bb1_ref[...]), 0.0)
            h2t_s[:, pl.ds(j * bd, bd)] = h2
            xwnt = jax.lax.dot(
                wnt_ref[...].astype(jnp.bfloat16), h2.astype(jnp.bfloat16),
                preferred_element_type=jnp.float32,
            ).astype(jnp.bfloat16)
            col = j * bd + jax.lax.broadcasted_iota(jnp.int32, (1, bd), 1)
            xwt3_s[:, pl.ds(j * bd, bd)] = jnp.where(
                col < n, xwnt, jnp.bfloat16(0))

        @pl.when(l == 1)
        def _():
            t = acc_ref[...] + b2_ref[...] + h2t_s[:, pl.ds(j * bd, bd)]
            h3 = jnp.maximum(_ln(t, g2_ref[...], bb2_ref[...]), 0.0)
            out_ref[...] = h3.T


def kernel(X, adj_mat, W0, b0, W1, b1, W2, b2, ln1_g, ln1_b, ln2_g, ln2_b):
    n, d = X.shape
    h = W0.shape[1]
    ks = 2048
    np_ = ((n + ks - 1) // ks) * ks
    num_k = np_ // ks

    bd1 = 1024
    num_j1 = pl.cdiv(n, bd1)
    h1t, adj_bf, xw2t = pl.pallas_call(
        functools.partial(_layer1_kernel, num_k, ks, n),
        grid=(num_j1, num_k),
        in_specs=[
            pl.BlockSpec((ks, bd1), lambda j, k: (k, j)),
            pl.BlockSpec((np_, d), lambda j, k: (0, 0)),
            pl.BlockSpec((d, h), lambda j, k: (0, 0)),
            pl.BlockSpec((h, 1), lambda j, k: (0, 0)),
            pl.BlockSpec((h, h), lambda j, k: (0, 0)),
        ],
        out_specs=[
            pl.BlockSpec((h, bd1), lambda j, k: (0, j)),
            pl.BlockSpec((ks, bd1), lambda j, k: (k, j)),
            pl.BlockSpec((h, bd1), lambda j, k: (0, j)),
        ],
        out_shape=[
            jax.ShapeDtypeStruct((h, n), jnp.float32),
            jax.ShapeDtypeStruct((np_, n), jnp.bfloat16),
            jax.ShapeDtypeStruct((h, np_), jnp.bfloat16),
        ],
        scratch_shapes=[pltpu.VMEM((h, bd1), jnp.float32),
                        pltpu.VMEM((h, np_), jnp.bfloat16)],
        compiler_params=pltpu.CompilerParams(
            dimension_semantics=("parallel", "arbitrary")),
    )(adj_mat, X, W0, b0.reshape(h, 1), W1.T)

    bd = 2048
    num_j = pl.cdiv(n, bd)
    out, = pl.pallas_call(
        functools.partial(_layer23_kernel, num_k, ks, n),
        grid=(2, num_j, num_k),
        in_specs=[
            pl.BlockSpec((ks, bd), lambda l, j, k: (k, j)),
            pl.BlockSpec((h, np_), lambda l, j, k: (0, 0)),
            pl.BlockSpec((h, bd),
                         lambda l, j, k: (0, jnp.where(l == 0, j, 0))),
            pl.BlockSpec((h, 1), lambda l, j, k: (0, 0)),
            pl.BlockSpec((h, 1), lambda l, j, k: (0, 0)),
            pl.BlockSpec((h, 1), lambda l, j, k: (0, 0)),
            pl.BlockSpec((h, 1), lambda l, j, k: (0, 0)),
            pl.BlockSpec((h, 1), lambda l, j, k: (0, 0)),
            pl.BlockSpec((h, 1), lambda l, j, k: (0, 0)),
            pl.BlockSpec((h, h), lambda l, j, k: (0, 0)),
        ],
        out_specs=[
            pl.BlockSpec((bd, h),
                         lambda l, j, k: (jnp.where(l == 1, j, 0), 0)),
        ],
        out_shape=[jax.ShapeDtypeStruct((n, h), jnp.float32)],
        scratch_shapes=[pltpu.VMEM((h, bd), jnp.float32),
                        pltpu.VMEM((h, np_), jnp.bfloat16),
                        pltpu.VMEM((h, np_), jnp.float32)],
        compiler_params=pltpu.CompilerParams(
            dimension_semantics=("arbitrary", "arbitrary", "arbitrary")),
    )(adj_bf, xw2t, h1t,
      b1.reshape(h, 1), ln1_g.reshape(h, 1), ln1_b.reshape(h, 1),
      b2.reshape(h, 1), ln2_g.reshape(h, 1), ln2_b.reshape(h, 1), W2.T)
    return out


# speedup vs baseline: 62.1798x; 1.0019x over previous
import functools

import jax
import jax.numpy as jnp
from jax.experimental import pallas as pl
from jax.experimental.pallas import tpu as pltpu


def _layer1_kernel(num_k, ks, n, adj_ref, x_ref, w0_ref, b_ref, wnt_ref,
                   ht_ref, abf_ref, xwnt_ref, acc_ref, xwt_s):
    j = pl.program_id(0)
    k = pl.program_id(1)
    np_ = num_k * ks

    @pl.when((j == 0) & (k == 0))
    def _():
        v = jax.lax.dot_general(
            w0_ref[...], x_ref[...], (((0,), (1,)), ((), ())),
            preferred_element_type=jnp.float32,
        )
        col = jax.lax.broadcasted_iota(jnp.int32, (1, np_), 1)
        xwt_s[...] = jnp.where(col < n, v.astype(jnp.bfloat16),
                               jnp.bfloat16(0))

    @pl.when(k == 0)
    def _():
        acc_ref[...] = jnp.zeros_like(acc_ref)

    a = adj_ref[...].astype(jnp.bfloat16)
    rows = k * ks + jax.lax.broadcasted_iota(jnp.int32, (ks, 1), 0)
    a = jnp.where(rows < n, a, jnp.bfloat16(0))
    abf_ref[...] = a
    acc_ref[...] += jax.lax.dot(
        xwt_s[:, pl.ds(k * ks, ks)], a, preferred_element_type=jnp.float32)

    @pl.when(k == num_k - 1)
    def _():
        ht = jnp.maximum(acc_ref[...] + b_ref[...], 0.0)
        ht_ref[...] = ht
        bd = ht_ref.shape[1]
        xwnt = jax.lax.dot(
            wnt_ref[...].astype(jnp.bfloat16), ht.astype(jnp.bfloat16),
            preferred_element_type=jnp.float32,
        ).astype(jnp.bfloat16)
        col = j * bd + jax.lax.broadcasted_iota(jnp.int32, (1, bd), 1)
        xwnt_ref[...] = jnp.where(col < n, xwnt, jnp.bfloat16(0))


def _ln(t, g, bb):
    mu = jnp.mean(t, axis=0, keepdims=True)
    var = jnp.mean((t - mu) ** 2, axis=0, keepdims=True)
    return (t - mu) * jax.lax.rsqrt(var + 1e-5) * g + bb


def _layer23_kernel(num_k, ks, n, abf_ref, xwt_ref, rest_ref,
                    b1_ref, g1_ref, bb1_ref, b2_ref, g2_ref, bb2_ref, wnt_ref,
                    out_ref, acc_ref, xwt3_s, h2t_s):
    l = pl.program_id(0)
    j = pl.program_id(1)
    k = pl.program_id(2)
    bd = acc_ref.shape[1]

    @pl.when(k == 0)
    def _():
        acc_ref[...] = jnp.zeros_like(acc_ref)

    a = abf_ref[...]

    @pl.when(l == 0)
    def _():
        acc_ref[...] += jax.lax.dot(
            xwt_ref[:, pl.ds(k * ks, ks)], a,
            preferred_element_type=jnp.float32)

    @pl.when(l == 1)
    def _():
        acc_ref[...] += jax.lax.dot(
            xwt3_s[:, pl.ds(k * ks, ks)], a,
            preferred_element_type=jnp.float32)

    @pl.when(k == num_k - 1)
    def _():
        @pl.when(l == 0)
        def _():
            t = acc_ref[...] + b1_ref[...] + rest_ref[...]
            h2 = jnp.maximum(_ln(t, g1_ref[...], bb1_ref[...]), 0.0)
            h2t_s[:, pl.ds(j * bd, bd)] = h2
            xwnt = jax.lax.dot(
                wnt_ref[...].astype(jnp.bfloat16), h2.astype(jnp.bfloat16),
                preferred_element_type=jnp.float32,
            ).astype(jnp.bfloat16)
            col = j * bd + jax.lax.broadcasted_iota(jnp.int32, (1, bd), 1)
            xwt3_s[:, pl.ds(j * bd, bd)] = jnp.where(
                col < n, xwnt, jnp.bfloat16(0))

        @pl.when(l == 1)
        def _():
            t = acc_ref[...] + b2_ref[...] + h2t_s[:, pl.ds(j * bd, bd)]
            h3 = jnp.maximum(_ln(t, g2_ref[...], bb2_ref[...]), 0.0)
            out_ref[...] = h3.T


def kernel(X, adj_mat, W0, b0, W1, b1, W2, b2, ln1_g, ln1_b, ln2_g, ln2_b):
    n, d = X.shape
    h = W0.shape[1]
    ks = 2048
    np_ = ((n + ks - 1) // ks) * ks
    num_k = np_ // ks

    ks1 = 1024
    num_k1 = np_ // ks1
    bd1 = 2048
    num_j1 = pl.cdiv(n, bd1)
    h1t, adj_bf, xw2t = pl.pallas_call(
        functools.partial(_layer1_kernel, num_k1, ks1, n),
        grid=(num_j1, num_k1),
        in_specs=[
            pl.BlockSpec((ks1, bd1), lambda j, k: (k, j)),
            pl.BlockSpec((np_, d), lambda j, k: (0, 0)),
            pl.BlockSpec((d, h), lambda j, k: (0, 0)),
            pl.BlockSpec((h, 1), lambda j, k: (0, 0)),
            pl.BlockSpec((h, h), lambda j, k: (0, 0)),
        ],
        out_specs=[
            pl.BlockSpec((h, bd1), lambda j, k: (0, j)),
            pl.BlockSpec((ks1, bd1), lambda j, k: (k, j)),
            pl.BlockSpec((h, bd1), lambda j, k: (0, j)),
        ],
        out_shape=[
            jax.ShapeDtypeStruct((h, n), jnp.float32),
            jax.ShapeDtypeStruct((np_, n), jnp.bfloat16),
            jax.ShapeDtypeStruct((h, np_), jnp.bfloat16),
        ],
        scratch_shapes=[pltpu.VMEM((h, bd1), jnp.float32),
                        pltpu.VMEM((h, np_), jnp.bfloat16)],
        compiler_params=pltpu.CompilerParams(
            dimension_semantics=("parallel", "arbitrary")),
    )(adj_mat, X, W0, b0.reshape(h, 1), W1.T)

    bd = 2048
    num_j = pl.cdiv(n, bd)
    out, = pl.pallas_call(
        functools.partial(_layer23_kernel, num_k, ks, n),
        grid=(2, num_j, num_k),
        in_specs=[
            pl.BlockSpec((ks, bd), lambda l, j, k: (k, j)),
            pl.BlockSpec((h, np_), lambda l, j, k: (0, 0)),
            pl.BlockSpec((h, bd),
                         lambda l, j, k: (0, jnp.where(l == 0, j, 0))),
            pl.BlockSpec((h, 1), lambda l, j, k: (0, 0)),
            pl.BlockSpec((h, 1), lambda l, j, k: (0, 0)),
            pl.BlockSpec((h, 1), lambda l, j, k: (0, 0)),
            pl.BlockSpec((h, 1), lambda l, j, k: (0, 0)),
            pl.BlockSpec((h, 1), lambda l, j, k: (0, 0)),
            pl.BlockSpec((h, 1), lambda l, j, k: (0, 0)),
            pl.BlockSpec((h, h), lambda l, j, k: (0, 0)),
        ],
        out_specs=[
            pl.BlockSpec((bd, h),
                         lambda l, j, k: (jnp.where(l == 1, j, 0), 0)),
        ],
        out_shape=[jax.ShapeDtypeStruct((n, h), jnp.float32)],
        scratch_shapes=[pltpu.VMEM((h, bd), jnp.float32),
                        pltpu.VMEM((h, np_), jnp.bfloat16),
                        pltpu.VMEM((h, np_), jnp.float32)],
        compiler_params=pltpu.CompilerParams(
            dimension_semantics=("arbitrary", "arbitrary", "arbitrary")),
    )(adj_bf, xw2t, h1t,
      b1.reshape(h, 1), ln1_g.reshape(h, 1), ln1_b.reshape(h, 1),
      b2.reshape(h, 1), ln2_g.reshape(h, 1), ln2_b.reshape(h, 1), W2.T)
    return out


# speedup vs baseline: 62.3181x; 1.0022x over previous
import functools

import jax
import jax.numpy as jnp
from jax.experimental import pallas as pl
from jax.experimental.pallas import tpu as pltpu


def _layer1_kernel(num_k, ks, n, adj_ref, x_ref, w0_ref, b_ref, wnt_ref,
                   ht_ref, abf_ref, xwnt_ref, acc_ref, xwt_s):
    j = pl.program_id(0)
    k = pl.program_id(1)
    np_ = num_k * ks

    @pl.when((j == 0) & (k == 0))
    def _():
        v = jax.lax.dot_general(
            w0_ref[...], x_ref[...], (((0,), (1,)), ((), ())),
            preferred_element_type=jnp.float32,
        )
        col = jax.lax.broadcasted_iota(jnp.int32, (1, np_), 1)
        xwt_s[...] = jnp.where(col < n, v.astype(jnp.bfloat16),
                               jnp.bfloat16(0))

    @pl.when(k == 0)
    def _():
        acc_ref[...] = jnp.zeros_like(acc_ref)

    a = adj_ref[...].astype(jnp.bfloat16)
    rows = k * ks + jax.lax.broadcasted_iota(jnp.int32, (ks, 1), 0)
    a = jnp.where(rows < n, a, jnp.bfloat16(0))
    abf_ref[...] = a
    acc_ref[...] += jax.lax.dot(
        xwt_s[:, pl.ds(k * ks, ks)], a, preferred_element_type=jnp.float32)

    @pl.when(k == num_k - 1)
    def _():
        ht = jnp.maximum(acc_ref[...] + b_ref[...], 0.0)
        ht_ref[...] = ht.astype(jnp.bfloat16)
        bd = ht_ref.shape[1]
        xwnt = jax.lax.dot(
            wnt_ref[...].astype(jnp.bfloat16), ht.astype(jnp.bfloat16),
            preferred_element_type=jnp.float32,
        ).astype(jnp.bfloat16)
        col = j * bd + jax.lax.broadcasted_iota(jnp.int32, (1, bd), 1)
        xwnt_ref[...] = jnp.where(col < n, xwnt, jnp.bfloat16(0))


def _ln(t, g, bb):
    mu = jnp.mean(t, axis=0, keepdims=True)
    var = jnp.mean((t - mu) ** 2, axis=0, keepdims=True)
    return (t - mu) * jax.lax.rsqrt(var + 1e-5) * g + bb


def _layer23_kernel(num_k, ks, n, abf_ref, xwt_ref, rest_ref,
                    b1_ref, g1_ref, bb1_ref, b2_ref, g2_ref, bb2_ref, wnt_ref,
                    out_ref, acc_ref, xwt3_s, h2t_s):
    l = pl.program_id(0)
    j = pl.program_id(1)
    k = pl.program_id(2)
    bd = acc_ref.shape[1]

    @pl.when(k == 0)
    def _():
        acc_ref[...] = jnp.zeros_like(acc_ref)

    a = abf_ref[...]

    @pl.when(l == 0)
    def _():
        acc_ref[...] += jax.lax.dot(
            xwt_ref[:, pl.ds(k * ks, ks)], a,
            preferred_element_type=jnp.float32)

    @pl.when(l == 1)
    def _():
        acc_ref[...] += jax.lax.dot(
            xwt3_s[:, pl.ds(k * ks, ks)], a,
            preferred_element_type=jnp.float32)

    @pl.when(k == num_k - 1)
    def _():
        @pl.when(l == 0)
        def _():
            t = acc_ref[...] + b1_ref[...] + rest_ref[...].astype(jnp.float32)
            h2 = jnp.maximum(_ln(t, g1_ref[...], bb1_ref[...]), 0.0)
            h2t_s[:, pl.ds(j * bd, bd)] = h2
            xwnt = jax.lax.dot(
                wnt_ref[...].astype(jnp.bfloat16), h2.astype(jnp.bfloat16),
                preferred_element_type=jnp.float32,
            ).astype(jnp.bfloat16)
            col = j * bd + jax.lax.broadcasted_iota(jnp.int32, (1, bd), 1)
            xwt3_s[:, pl.ds(j * bd, bd)] = jnp.where(
                col < n, xwnt, jnp.bfloat16(0))

        @pl.when(l == 1)
        def _():
            t = acc_ref[...] + b2_ref[...] + h2t_s[:, pl.ds(j * bd, bd)]
            h3 = jnp.maximum(_ln(t, g2_ref[...], bb2_ref[...]), 0.0)
            out_ref[...] = h3.T


def kernel(X, adj_mat, W0, b0, W1, b1, W2, b2, ln1_g, ln1_b, ln2_g, ln2_b):
    n, d = X.shape
    h = W0.shape[1]
    ks = 2048
    np_ = ((n + ks - 1) // ks) * ks
    num_k = np_ // ks

    ks1 = 1024
    num_k1 = np_ // ks1
    bd1 = 2048
    num_j1 = pl.cdiv(n, bd1)
    h1t, adj_bf, xw2t = pl.pallas_call(
        functools.partial(_layer1_kernel, num_k1, ks1, n),
        grid=(num_j1, num_k1),
        in_specs=[
            pl.BlockSpec((ks1, bd1), lambda j, k: (k, j)),
            pl.BlockSpec((np_, d), lambda j, k: (0, 0)),
            pl.BlockSpec((d, h), lambda j, k: (0, 0)),
            pl.BlockSpec((h, 1), lambda j, k: (0, 0)),
            pl.BlockSpec((h, h), lambda j, k: (0, 0)),
        ],
        out_specs=[
            pl.BlockSpec((h, bd1), lambda j, k: (0, j)),
            pl.BlockSpec((ks1, bd1), lambda j, k: (k, j)),
            pl.BlockSpec((h, bd1), lambda j, k: (0, j)),
        ],
        out_shape=[
            jax.ShapeDtypeStruct((h, n), jnp.bfloat16),
            jax.ShapeDtypeStruct((np_, n), jnp.bfloat16),
            jax.ShapeDtypeStruct((h, np_), jnp.bfloat16),
        ],
        scratch_shapes=[pltpu.VMEM((h, bd1), jnp.float32),
                        pltpu.VMEM((h, np_), jnp.bfloat16)],
        compiler_params=pltpu.CompilerParams(
            dimension_semantics=("parallel", "arbitrary")),
    )(adj_mat, X, W0, b0.reshape(h, 1), W1.T)

    bd = 2048
    num_j = pl.cdiv(n, bd)
    out, = pl.pallas_call(
        functools.partial(_layer23_kernel, num_k, ks, n),
        grid=(2, num_j, num_k),
        in_specs=[
            pl.BlockSpec((ks, bd), lambda l, j, k: (k, j)),
            pl.BlockSpec((h, np_), lambda l, j, k: (0, 0)),
            pl.BlockSpec((h, bd),
                         lambda l, j, k: (0, jnp.where(l == 0, j, 0))),
            pl.BlockSpec((h, 1), lambda l, j, k: (0, 0)),
            pl.BlockSpec((h, 1), lambda l, j, k: (0, 0)),
            pl.BlockSpec((h, 1), lambda l, j, k: (0, 0)),
            pl.BlockSpec((h, 1), lambda l, j, k: (0, 0)),
            pl.BlockSpec((h, 1), lambda l, j, k: (0, 0)),
            pl.BlockSpec((h, 1), lambda l, j, k: (0, 0)),
            pl.BlockSpec((h, h), lambda l, j, k: (0, 0)),
        ],
        out_specs=[
            pl.BlockSpec((bd, h),
                         lambda l, j, k: (jnp.where(l == 1, j, 0), 0)),
        ],
        out_shape=[jax.ShapeDtypeStruct((n, h), jnp.float32)],
        scratch_shapes=[pltpu.VMEM((h, bd), jnp.float32),
                        pltpu.VMEM((h, np_), jnp.bfloat16),
                        pltpu.VMEM((h, np_), jnp.float32)],
        compiler_params=pltpu.CompilerParams(
            dimension_semantics=("arbitrary", "arbitrary", "arbitrary")),
    )(adj_bf, xw2t, h1t,
      b1.reshape(h, 1), ln1_g.reshape(h, 1), ln1_b.reshape(h, 1),
      b2.reshape(h, 1), ln2_g.reshape(h, 1), ln2_b.reshape(h, 1), W2.T)
    return out


# speedup vs baseline: 62.4071x; 1.0014x over previous
import functools

import jax
import jax.numpy as jnp
from jax.experimental import pallas as pl
from jax.experimental.pallas import tpu as pltpu


def _layer1_kernel(num_k, ks, n, adj_ref, x_ref, w0_ref, b_ref, wnt_ref,
                   ht_ref, abf_ref, xwnt_ref, acc_ref, xwt_s):
    j = pl.program_id(0)
    k = pl.program_id(1)
    np_ = num_k * ks

    @pl.when((j == 0) & (k == 0))
    def _():
        v = jax.lax.dot_general(
            w0_ref[...], x_ref[...], (((0,), (1,)), ((), ())),
            preferred_element_type=jnp.float32,
        )
        col = jax.lax.broadcasted_iota(jnp.int32, (1, np_), 1)
        xwt_s[...] = jnp.where(col < n, v.astype(jnp.bfloat16),
                               jnp.bfloat16(0))

    @pl.when(k == 0)
    def _():
        acc_ref[...] = jnp.zeros_like(acc_ref)

    a = adj_ref[...].astype(jnp.bfloat16)
    rows = k * ks + jax.lax.broadcasted_iota(jnp.int32, (ks, 1), 0)
    a = jnp.where(rows < n, a, jnp.bfloat16(0))
    abf_ref[...] = a
    acc_ref[...] += jax.lax.dot(
        xwt_s[:, pl.ds(k * ks, ks)], a, preferred_element_type=jnp.float32)

    @pl.when(k == num_k - 1)
    def _():
        ht = jnp.maximum(acc_ref[...] + b_ref[...], 0.0)
        ht_ref[...] = ht.astype(jnp.bfloat16)
        bd = ht_ref.shape[1]
        xwnt = jax.lax.dot(
            wnt_ref[...].astype(jnp.bfloat16), ht.astype(jnp.bfloat16),
            preferred_element_type=jnp.float32,
        ).astype(jnp.bfloat16)
        col = j * bd + jax.lax.broadcasted_iota(jnp.int32, (1, bd), 1)
        xwnt_ref[...] = jnp.where(col < n, xwnt, jnp.bfloat16(0))


def _ln(t, g, bb):
    mu = jnp.mean(t, axis=0, keepdims=True)
    var = jnp.mean((t - mu) ** 2, axis=0, keepdims=True)
    return (t - mu) * jax.lax.rsqrt(var + 1e-5) * g + bb


def _layer23_kernel(num_k, ks, n, abf_ref, xwt_ref, rest_ref,
                    b1_ref, g1_ref, bb1_ref, b2_ref, g2_ref, bb2_ref, wnt_ref,
                    out_ref, acc_ref, xwt3_s, h2t_s):
    l = pl.program_id(0)
    j = pl.program_id(1)
    k = pl.program_id(2)
    bd = acc_ref.shape[1]

    @pl.when(k == 0)
    def _():
        acc_ref[...] = jnp.zeros_like(acc_ref)

    a = abf_ref[...]

    @pl.when(l == 0)
    def _():
        acc_ref[...] += jax.lax.dot(
            xwt_ref[:, pl.ds(k * ks, ks)], a,
            preferred_element_type=jnp.float32)

    @pl.when(l == 1)
    def _():
        acc_ref[...] += jax.lax.dot(
            xwt3_s[:, pl.ds(k * ks, ks)], a,
            preferred_element_type=jnp.float32)

    @pl.when(k == num_k - 1)
    def _():
        @pl.when(l == 0)
        def _():
            t = acc_ref[...] + b1_ref[...] + rest_ref[...].astype(jnp.float32)
            h2 = jnp.maximum(_ln(t, g1_ref[...], bb1_ref[...]), 0.0)
            h2t_s[:, pl.ds(j * bd, bd)] = h2
            xwnt = jax.lax.dot(
                wnt_ref[...].astype(jnp.bfloat16), h2.astype(jnp.bfloat16),
                preferred_element_type=jnp.float32,
            ).astype(jnp.bfloat16)
            col = j * bd + jax.lax.broadcasted_iota(jnp.int32, (1, bd), 1)
            xwt3_s[:, pl.ds(j * bd, bd)] = jnp.where(
                col < n, xwnt, jnp.bfloat16(0))

        @pl.when(l == 1)
        def _():
            t = acc_ref[...] + b2_ref[...] + h2t_s[:, pl.ds(j * bd, bd)]
            h3 = jnp.maximum(_ln(t, g2_ref[...], bb2_ref[...]), 0.0)
            out_ref[...] = h3.T


def kernel(X, adj_mat, W0, b0, W1, b1, W2, b2, ln1_g, ln1_b, ln2_g, ln2_b):
    n, d = X.shape
    h = W0.shape[1]
    ks = 2048
    np_ = ((n + ks - 1) // ks) * ks
    num_k = np_ // ks

    ks1 = 1024
    num_k1 = np_ // ks1
    bd1 = 2048
    num_j1 = pl.cdiv(n, bd1)
    h1t, adj_bf, xw2t = pl.pallas_call(
        functools.partial(_layer1_kernel, num_k1, ks1, n),
        grid=(num_j1, num_k1),
        in_specs=[
            pl.BlockSpec((ks1, bd1), lambda j, k: (k, j)),
            pl.BlockSpec((np_, d), lambda j, k: (0, 0)),
            pl.BlockSpec((d, h), lambda j, k: (0, 0)),
            pl.BlockSpec((h, 1), lambda j, k: (0, 0)),
            pl.BlockSpec((h, h), lambda j, k: (0, 0)),
        ],
        out_specs=[
            pl.BlockSpec((h, bd1), lambda j, k: (0, j)),
            pl.BlockSpec((ks1, bd1), lambda j, k: (k, j)),
            pl.BlockSpec((h, bd1), lambda j, k: (0, j)),
        ],
        out_shape=[
            jax.ShapeDtypeStruct((h, n), jnp.bfloat16),
            jax.ShapeDtypeStruct((np_, n), jnp.bfloat16),
            jax.ShapeDtypeStruct((h, np_), jnp.bfloat16),
        ],
        scratch_shapes=[pltpu.VMEM((h, bd1), jnp.float32),
                        pltpu.VMEM((h, np_), jnp.bfloat16)],
        compiler_params=pltpu.CompilerParams(
            dimension_semantics=("arbitrary", "arbitrary")),
    )(adj_mat, X, W0, b0.reshape(h, 1), W1.T)

    bd = 2048
    num_j = pl.cdiv(n, bd)
    out, = pl.pallas_call(
        functools.partial(_layer23_kernel, num_k, ks, n),
        grid=(2, num_j, num_k),
        in_specs=[
            pl.BlockSpec((ks, bd), lambda l, j, k: (k, j)),
            pl.BlockSpec((h, np_), lambda l, j, k: (0, 0)),
            pl.BlockSpec((h, bd),
                         lambda l, j, k: (0, jnp.where(l == 0, j, 0))),
            pl.BlockSpec((h, 1), lambda l, j, k: (0, 0)),
            pl.BlockSpec((h, 1), lambda l, j, k: (0, 0)),
            pl.BlockSpec((h, 1), lambda l, j, k: (0, 0)),
            pl.BlockSpec((h, 1), lambda l, j, k: (0, 0)),
            pl.BlockSpec((h, 1), lambda l, j, k: (0, 0)),
            pl.BlockSpec((h, 1), lambda l, j, k: (0, 0)),
            pl.BlockSpec((h, h), lambda l, j, k: (0, 0)),
        ],
        out_specs=[
            pl.BlockSpec((bd, h),
                         lambda l, j, k: (jnp.where(l == 1, j, 0), 0)),
        ],
        out_shape=[jax.ShapeDtypeStruct((n, h), jnp.float32)],
        scratch_shapes=[pltpu.VMEM((h, bd), jnp.float32),
                        pltpu.VMEM((h, np_), jnp.bfloat16),
                        pltpu.VMEM((h, np_), jnp.float32)],
        compiler_params=pltpu.CompilerParams(
            dimension_semantics=("arbitrary", "arbitrary", "arbitrary")),
    )(adj_bf, xw2t, h1t,
      b1.reshape(h, 1), ln1_g.reshape(h, 1), ln1_b.reshape(h, 1),
      b2.reshape(h, 1), ln2_g.reshape(h, 1), ln2_b.reshape(h, 1), W2.T)
    return out
